# initial kernel scaffold (unmeasured)
import jax
import jax.numpy as jnp
from jax import lax
from jax.experimental import pallas as pl
from jax.experimental.pallas import tpu as pltpu


def kernel(
    x,
):
    def body(*refs):
        pass

    out_shape = jax.ShapeDtypeStruct(..., jnp.float32)
    return pl.pallas_call(body, out_shape=out_shape)(...)


# baseline (device time: 18407 ns/iter reference)
import jax
import jax.numpy as jnp
from jax import lax
from jax.experimental import pallas as pl
from jax.experimental.pallas import tpu as pltpu


def kernel(x):
    m, n = x.shape
    half = n // 2
    out_dtype = jnp.bfloat16

    def body(x_ref, out_ref, send_buf, send_sem, recv_sem):
        my_x = lax.axis_index("x")
        my_y = lax.axis_index("y")
        peer = (my_x, 1 - my_y)

        barrier_sem = pltpu.get_barrier_semaphore()
        pl.semaphore_signal(
            barrier_sem, inc=1,
            device_id=peer, device_id_type=pl.DeviceIdType.MESH,
        )
        pl.semaphore_wait(barrier_sem, 1)

        @pl.when(my_y == 0)
        def _():
            send_buf[...] = x_ref[:, half:].astype(out_dtype)

        @pl.when(my_y == 1)
        def _():
            send_buf[...] = x_ref[:, :half].astype(out_dtype)

        rdma = pltpu.make_async_remote_copy(
            src_ref=send_buf,
            dst_ref=out_ref.at[pl.ds(my_y * m, m)],
            send_sem=send_sem,
            recv_sem=recv_sem,
            device_id=peer,
            device_id_type=pl.DeviceIdType.MESH,
        )
        rdma.start()

        @pl.when(my_y == 0)
        def _():
            out_ref[pl.ds(0, m), :] = x_ref[:, :half].astype(out_dtype)

        @pl.when(my_y == 1)
        def _():
            out_ref[pl.ds(m, m), :] = x_ref[:, half:].astype(out_dtype)

        rdma.wait()

    return pl.pallas_call(
        body,
        out_shape=jax.ShapeDtypeStruct((2 * m, half), out_dtype),
        in_specs=[pl.BlockSpec(memory_space=pltpu.VMEM)],
        out_specs=pl.BlockSpec(memory_space=pltpu.VMEM),
        scratch_shapes=[
            pltpu.VMEM((m, half), out_dtype),
            pltpu.SemaphoreType.DMA,
            pltpu.SemaphoreType.DMA,
        ],
        compiler_params=pltpu.CompilerParams(collective_id=0),
    )(x)


# device time: 16700 ns/iter; 1.1022x vs baseline; 1.1022x over previous
import jax
import jax.numpy as jnp
from jax import lax
from jax.experimental import pallas as pl
from jax.experimental.pallas import tpu as pltpu

C = 4


def kernel(x):
    m, n = x.shape
    half = n // 2
    hrows = m // 2
    chs = hrows // C
    out_dtype = jnp.bfloat16

    def body(x_ref, out_ref, send_buf, ysend_sems, yrecv_sems,
             xsend_sems, xrecv_sems):
        my_x = lax.axis_index("x")
        my_y = lax.axis_index("y")
        peer_y = (my_x, 1 - my_y)
        peer_x = (1 - my_x, my_y)

        barrier_sem = pltpu.get_barrier_semaphore()
        for nbr in (peer_y, peer_x):
            pl.semaphore_signal(
                barrier_sem, inc=1,
                device_id=nbr, device_id_type=pl.DeviceIdType.MESH,
            )
        pl.semaphore_wait(barrier_sem, 2)

        my_half_off = my_x * hrows

        rdmas_y = []
        for c in range(C):
            row0 = my_half_off + c * chs

            @pl.when(my_y == 0)
            def _(row0=row0, c=c):
                send_buf[pl.ds(c * chs, chs), :] = (
                    x_ref[pl.ds(row0, chs), half:].astype(out_dtype))

            @pl.when(my_y == 1)
            def _(row0=row0, c=c):
                send_buf[pl.ds(c * chs, chs), :] = (
                    x_ref[pl.ds(row0, chs), :half].astype(out_dtype))

            rdma = pltpu.make_async_remote_copy(
                src_ref=send_buf.at[pl.ds(c * chs, chs)],
                dst_ref=out_ref.at[pl.ds(my_y * m + row0, chs)],
                send_sem=ysend_sems.at[c],
                recv_sem=yrecv_sems.at[c],
                device_id=peer_y,
                device_id_type=pl.DeviceIdType.MESH,
            )
            rdma.start()
            rdmas_y.append(rdma)

        @pl.when(my_y == 0)
        def _():
            out_ref[pl.ds(0, m), :] = x_ref[:, :half].astype(out_dtype)

        @pl.when(my_y == 1)
        def _():
            out_ref[pl.ds(m, m), :] = x_ref[:, half:].astype(out_dtype)

        rdmas_x = []
        for c in range(C):
            rdmas_y[c].wait_recv()
            row0 = (1 - my_y) * m + my_half_off + c * chs
            fwd = pltpu.make_async_remote_copy(
                src_ref=out_ref.at[pl.ds(row0, chs)],
                dst_ref=out_ref.at[pl.ds(row0, chs)],
                send_sem=xsend_sems.at[c],
                recv_sem=xrecv_sems.at[c],
                device_id=peer_x,
                device_id_type=pl.DeviceIdType.MESH,
            )
            fwd.start()
            rdmas_x.append(fwd)

        for c in range(C):
            rdmas_x[c].wait_recv()
        for c in range(C):
            rdmas_y[c].wait_send()
            rdmas_x[c].wait_send()

    return pl.pallas_call(
        body,
        out_shape=jax.ShapeDtypeStruct((2 * m, half), out_dtype),
        in_specs=[pl.BlockSpec(memory_space=pltpu.VMEM)],
        out_specs=pl.BlockSpec(memory_space=pltpu.VMEM),
        scratch_shapes=[
            pltpu.VMEM((hrows, half), out_dtype),
            pltpu.SemaphoreType.DMA((C,)),
            pltpu.SemaphoreType.DMA((C,)),
            pltpu.SemaphoreType.DMA((C,)),
            pltpu.SemaphoreType.DMA((C,)),
        ],
        compiler_params=pltpu.CompilerParams(collective_id=0),
    )(x)


# device time: 16162 ns/iter; 1.1389x vs baseline; 1.0333x over previous
import jax
import jax.numpy as jnp
from jax import lax
from jax.experimental import pallas as pl
from jax.experimental.pallas import tpu as pltpu

C = 8


def kernel(x):
    m, n = x.shape
    half = n // 2
    hrows = m // 2
    chs = hrows // C
    out_dtype = jnp.bfloat16

    def body(x_ref, out_ref, send_buf, ysend_sems, yrecv_sems,
             xsend_sems, xrecv_sems):
        my_x = lax.axis_index("x")
        my_y = lax.axis_index("y")
        peer_y = (my_x, 1 - my_y)
        peer_x = (1 - my_x, my_y)

        barrier_sem = pltpu.get_barrier_semaphore()
        for nbr in (peer_y, peer_x):
            pl.semaphore_signal(
                barrier_sem, inc=1,
                device_id=nbr, device_id_type=pl.DeviceIdType.MESH,
            )
        pl.semaphore_wait(barrier_sem, 2)

        my_half_off = my_x * hrows

        rdmas_y = []
        for c in range(C):
            row0 = my_half_off + c * chs

            @pl.when(my_y == 0)
            def _(row0=row0, c=c):
                send_buf[pl.ds(c * chs, chs), :] = (
                    x_ref[pl.ds(row0, chs), half:].astype(out_dtype))

            @pl.when(my_y == 1)
            def _(row0=row0, c=c):
                send_buf[pl.ds(c * chs, chs), :] = (
                    x_ref[pl.ds(row0, chs), :half].astype(out_dtype))

            rdma = pltpu.make_async_remote_copy(
                src_ref=send_buf.at[pl.ds(c * chs, chs)],
                dst_ref=out_ref.at[pl.ds(my_y * m + row0, chs)],
                send_sem=ysend_sems.at[c],
                recv_sem=yrecv_sems.at[c],
                device_id=peer_y,
                device_id_type=pl.DeviceIdType.MESH,
            )
            rdma.start()
            rdmas_y.append(rdma)

        @pl.when(my_y == 0)
        def _():
            out_ref[pl.ds(0, m), :] = x_ref[:, :half].astype(out_dtype)

        @pl.when(my_y == 1)
        def _():
            out_ref[pl.ds(m, m), :] = x_ref[:, half:].astype(out_dtype)

        rdmas_x = []
        for c in range(C):
            rdmas_y[c].wait_recv()
            row0 = (1 - my_y) * m + my_half_off + c * chs
            fwd = pltpu.make_async_remote_copy(
                src_ref=out_ref.at[pl.ds(row0, chs)],
                dst_ref=out_ref.at[pl.ds(row0, chs)],
                send_sem=xsend_sems.at[c],
                recv_sem=xrecv_sems.at[c],
                device_id=peer_x,
                device_id_type=pl.DeviceIdType.MESH,
            )
            fwd.start()
            rdmas_x.append(fwd)

        for c in range(C):
            rdmas_x[c].wait_recv()
        for c in range(C):
            rdmas_y[c].wait_send()
            rdmas_x[c].wait_send()

    return pl.pallas_call(
        body,
        out_shape=jax.ShapeDtypeStruct((2 * m, half), out_dtype),
        in_specs=[pl.BlockSpec(memory_space=pltpu.VMEM)],
        out_specs=pl.BlockSpec(memory_space=pltpu.VMEM),
        scratch_shapes=[
            pltpu.VMEM((hrows, half), out_dtype),
            pltpu.SemaphoreType.DMA((C,)),
            pltpu.SemaphoreType.DMA((C,)),
            pltpu.SemaphoreType.DMA((C,)),
            pltpu.SemaphoreType.DMA((C,)),
        ],
        compiler_params=pltpu.CompilerParams(collective_id=0),
    )(x)
